# Initial kernel scaffold; baseline (speedup 1.0000x reference)
import jax
import jax.numpy as jnp
from jax.experimental import pallas as pl


def kernel(edata0, edata1, edata2, dst0, dst1, dst2, features, r_vec, attn1_w, attn2_w, fus_w1, fus_b1, fus_w2):
    raise NotImplementedError("write your pallas kernel here")


# speedup vs baseline: 1.6190x; 1.6190x over previous
import functools

import jax
import jax.numpy as jnp
from jax.experimental import pallas as pl
from jax.experimental.pallas import tpu as pltpu

_N = 10000
_I = 160000
_L = 3
_HID = 128
_HEADS = 4
_P = 3
_ETYPES = ((0, 1), (2, 3), (4, 5))
_C = 640
_NC = _I // _C
_NB = 2000
_NBC = _N // _NB


def _build_rot_mats(r_vec):
    h2 = _HID // 2
    rv = r_vec / jnp.maximum(jnp.linalg.norm(r_vec, axis=2, keepdims=True), 1e-12)
    rv2 = jnp.stack((rv, rv), axis=1)
    rv2 = rv2.at[:, 1, :, 1].multiply(-1.0)
    rv2 = rv2.reshape(r_vec.shape[0] * 2, h2, 2)
    mats = []
    for p in range(_P):
        et = _ETYPES[p]
        finals = [None] * _L
        finals[_L - 1] = jnp.stack(
            [jnp.ones((h2,), jnp.float32), jnp.zeros((h2,), jnp.float32)], axis=1)
        for i in range(_L - 2, -1, -1):
            r = rv2[et[i]]
            prev = finals[i + 1]
            re = prev[:, 0] * r[:, 0] - prev[:, 1] * r[:, 1]
            im = prev[:, 0] * r[:, 1] + prev[:, 1] * r[:, 0]
            finals[i] = jnp.stack([re, im], axis=1)
        M = jnp.zeros((_L, _HID, _HID), jnp.float32)
        ks = jnp.arange(h2)
        for l in range(_L):
            fre = finals[l][:, 0] / _L
            fim = finals[l][:, 1] / _L
            M = M.at[l, 2 * ks, 2 * ks].set(fre)
            M = M.at[l, 2 * ks + 1, 2 * ks].set(-fim)
            M = M.at[l, 2 * ks, 2 * ks + 1].set(fim)
            M = M.at[l, 2 * ks + 1, 2 * ks + 1].set(fre)
        mats.append(M.reshape(_L * _HID, _HID))
    return jnp.stack(mats, axis=0)


def _agg_kernel(ed_ref, dst_ref, feat_ref, rot_ref, w1_ref, w2_ref,
                out_ref, a1_ref, denom_ref, emb_ref, a2_ref):
    p = pl.program_id(0)
    c = pl.program_id(1)

    @pl.when(c == 0)
    def _init():
        a1_ref[...] = jnp.dot(feat_ref[...], w1_ref[0].T,
                              preferred_element_type=jnp.float32)
        denom_ref[...] = jnp.zeros_like(denom_ref)
        out_ref[...] = jnp.zeros_like(out_ref)

    emb = jnp.dot(ed_ref[0], rot_ref[0], preferred_element_type=jnp.float32)
    emb_ref[...] = emb
    a2_ref[...] = jnp.dot(emb, w2_ref[0].T, preferred_element_type=jnp.float32)

    def body(i, _):
        n = dst_ref[p, i]
        a = a1_ref[pl.ds(n, 1), :] + a2_ref[pl.ds(i, 1), :]
        a = jnp.where(a >= 0, a, 0.01 * a)
        e = jnp.exp(a)
        denom_ref[pl.ds(n, 1), :] = denom_ref[pl.ds(n, 1), :] + e
        val = e.reshape(1, _HEADS, 1) * emb_ref[pl.ds(i, 1), :].reshape(1, 1, _HID)
        out_ref[0, pl.ds(n, 1)] = out_ref[0, pl.ds(n, 1)] + val
        return 0

    jax.lax.fori_loop(0, _C, body, 0)

    @pl.when(c == _NC - 1)
    def _finalize():
        hp = out_ref[0]
        dn = jnp.maximum(denom_ref[...], 1e-9)
        res = hp / dn[:, :, None]
        out_ref[0] = jnp.where(res > 0, res, jnp.exp(jnp.minimum(res, 0.0)) - 1.0)


def _fusion_kernel(h_ref, w1_ref, b1_ref, w2_ref, fused_ref, wacc_ref, beta_ref):
    phase = pl.program_id(0)
    c = pl.program_id(1)

    @pl.when(jnp.logical_and(phase == 0, c == 0))
    def _init():
        wacc_ref[...] = jnp.zeros_like(wacc_ref)

    @pl.when(phase == 0)
    def _accumulate():
        z = h_ref[...].reshape(_P * _NB, _HEADS * _HID)
        t = jnp.tanh(jnp.dot(z, w1_ref[...].T, preferred_element_type=jnp.float32)
                     + b1_ref[...])
        w = jnp.dot(t, w2_ref[...].T, preferred_element_type=jnp.float32)
        wacc_ref[...] = wacc_ref[...] + jnp.sum(w.reshape(_P, _NB, 1), axis=1)
        fused_ref[...] = jnp.zeros_like(fused_ref)

    @pl.when(phase == 1)
    def _fuse():
        wm = wacc_ref[...] / _N
        m = jnp.max(wm, axis=0, keepdims=True)
        e = jnp.exp(wm - m)
        beta = e / jnp.sum(e, axis=0, keepdims=True)
        beta_ref[...] = beta
        fused_ref[...] = jnp.sum(h_ref[...] * beta[:, :, None], axis=0)


@jax.jit
def kernel(edata0, edata1, edata2, dst0, dst1, dst2, features, r_vec,
           attn1_w, attn2_w, fus_w1, fus_b1, fus_w2):
    ed = jnp.stack([edata0.reshape(_I, _L * _HID),
                    edata1.reshape(_I, _L * _HID),
                    edata2.reshape(_I, _L * _HID)], axis=0)
    dst = jnp.stack([dst0, dst1, dst2], axis=0).astype(jnp.int32)
    rot = _build_rot_mats(r_vec)

    h = pl.pallas_call(
        _agg_kernel,
        grid=(_P, _NC),
        in_specs=[
            pl.BlockSpec((1, _C, _L * _HID), lambda p, c: (p, c, 0)),
            pl.BlockSpec((_P, _C), lambda p, c: (0, c), memory_space=pltpu.SMEM),
            pl.BlockSpec((_N, _HID), lambda p, c: (0, 0)),
            pl.BlockSpec((1, _L * _HID, _HID), lambda p, c: (p, 0, 0)),
            pl.BlockSpec((1, _HEADS, _HID), lambda p, c: (p, 0, 0)),
            pl.BlockSpec((1, _HEADS, _HID), lambda p, c: (p, 0, 0)),
        ],
        out_specs=pl.BlockSpec((1, _N, _HEADS, _HID), lambda p, c: (p, 0, 0, 0)),
        out_shape=jax.ShapeDtypeStruct((_P, _N, _HEADS, _HID), jnp.float32),
        scratch_shapes=[
            pltpu.VMEM((_N, _HEADS), jnp.float32),
            pltpu.VMEM((_N, _HEADS), jnp.float32),
            pltpu.VMEM((_C, _HID), jnp.float32),
            pltpu.VMEM((_C, _HEADS), jnp.float32),
        ],
    )(ed, dst, features, rot, attn1_w, attn2_w)

    hz = h.reshape(_P, _N, _HEADS * _HID)
    fused, wacc, beta = pl.pallas_call(
        _fusion_kernel,
        grid=(2, _NBC),
        in_specs=[
            pl.BlockSpec((_P, _NB, _HEADS * _HID), lambda ph, c: (0, c, 0)),
            pl.BlockSpec(fus_w1.shape, lambda ph, c: (0, 0)),
            pl.BlockSpec((1, fus_b1.shape[0]), lambda ph, c: (0, 0)),
            pl.BlockSpec(fus_w2.shape, lambda ph, c: (0, 0)),
        ],
        out_specs=[
            pl.BlockSpec((_NB, _HEADS * _HID), lambda ph, c: (c, 0)),
            pl.BlockSpec((_P, 1), lambda ph, c: (0, 0)),
            pl.BlockSpec((_P, 1), lambda ph, c: (0, 0)),
        ],
        out_shape=[
            jax.ShapeDtypeStruct((_N, _HEADS * _HID), jnp.float32),
            jax.ShapeDtypeStruct((_P, 1), jnp.float32),
            jax.ShapeDtypeStruct((_P, 1), jnp.float32),
        ],
    )(hz, fus_w1, fus_b1.reshape(1, -1), fus_w2)
    del wacc
    return fused, beta
